# baseline (speedup 1.0000x reference)
import jax
import jax.numpy as jnp
from jax import lax
from jax.experimental import pallas as pl
from jax.experimental.pallas import tpu as pltpu
from jax.experimental.pallas import tpu_sc as plsc

S = 2048
DH = 2048
DE = 1024
NE = 8
RBLK = 512
TBLK = 128
SBLK = 256
NSLOTS = 2 * S + NE * SBLK
NB = NSLOTS // SBLK
NBPAD = 32
NW = 32
NT = 16
APT = 2 * S // NT
GPT = NSLOTS // NW
CPT = 2 * S // NW
GCH = 8

_INTERPRET = False


def _silu(v):
    return v * jax.nn.sigmoid(v)


def _vsel(pred_scalar, a, b):
    pi = jnp.full((16,), pred_scalar.astype(jnp.int32))
    return jnp.where(pi == 1, a, b)


def _router_kernel(x_ref, wr_ref, w0_ref, w1_ref, e0_ref, e1_ref):
    xb = x_ref[...]
    lg = lax.dot_general(xb, wr_ref[...], (((1,), (1,)), ((), ())),
                         preferred_element_type=jnp.float32)
    m = jnp.max(lg, axis=-1, keepdims=True)
    p = jnp.exp(lg - m)
    sc = p / jnp.sum(p, axis=-1, keepdims=True)
    iota = lax.broadcasted_iota(jnp.int32, (RBLK, NE), 1)
    s0 = jnp.max(sc, axis=-1)
    a0 = jnp.min(jnp.where(sc == s0[:, None], iota, NE), axis=-1)
    sc1 = jnp.where(iota == a0[:, None], -1.0, sc)
    s1 = jnp.max(sc1, axis=-1)
    a1 = jnp.min(jnp.where(sc1 == s1[:, None], iota, NE), axis=-1)
    w0_ref[...] = s0
    w1_ref[...] = s1
    e0_ref[...] = a0
    e1_ref[...] = a1


def _shared_kernel(x_ref, wg_ref, wu_ref, wd_ref, out_ref):
    xb = x_ref[...]
    g = lax.dot_general(xb, wg_ref[...], (((1,), (1,)), ((), ())),
                        preferred_element_type=jnp.float32)
    u = lax.dot_general(xb, wu_ref[...], (((1,), (1,)), ((), ())),
                        preferred_element_type=jnp.float32)
    gu = _silu(g) * u
    out_ref[...] = lax.dot_general(gu, wd_ref[...], (((1,), (1,)), ((), ())),
                                   preferred_element_type=jnp.float32)


def _expert_kernel(be_ref, x_ref, wg_ref, wu_ref, wd_ref, sw_ref, y_ref):
    del be_ref
    xb = x_ref[...]
    g = lax.dot_general(xb, wg_ref[0], (((1,), (1,)), ((), ())),
                        preferred_element_type=jnp.float32)
    u = lax.dot_general(xb, wu_ref[0], (((1,), (1,)), ((), ())),
                        preferred_element_type=jnp.float32)
    h = _silu(g) * u
    y = lax.dot_general(h, wd_ref[0], (((1,), (1,)), ((), ())),
                        preferred_element_type=jnp.float32)
    y_ref[...] = y * sw_ref[...]


def _add3_kernel(a_ref, b_ref, c_ref, o_ref):
    o_ref[...] = a_ref[...] + b_ref[...] + c_ref[...]


def _pipelined_gather(src_hbm, out_hbm, idx_v, bufs, gsem, ssem,
                      base, n_chunks):
    def st_wait(j, off):
        pltpu.make_async_copy(bufs[j], out_hbm.at[pl.ds(off, GCH)],
                              ssem[j]).wait()

    def quad(q, carry):
        offs = [base + (q * 4 + j) * GCH for j in range(4)]
        gathers = []
        for j in range(4):
            @pl.when(q > 0)
            def _(j=j, off=offs[j]):
                st_wait(j, off)
            gathers.append(pltpu.async_copy(
                src_hbm.at[idx_v.at[pl.ds((q * 4 + j) * GCH, GCH)]],
                bufs[j], gsem[j]))
        for j in range(4):
            gathers[j].wait()
            pltpu.async_copy(bufs[j], out_hbm.at[pl.ds(offs[j], GCH)],
                             ssem[j])
        return carry

    lax.fori_loop(0, n_chunks // 4, quad, 0)
    for j in range(4):
        st_wait(j, base + (n_chunks - 4 + j) * GCH)


def _routing_dispatch_body(x_hbm, e0_hbm, e1_hbm, w0_hbm, w1_hbm,
                           xs_hbm, sw_hbm, pos_hbm, be_hbm,
                           ev0, wv0, posb, posb1d, tokb, wb,
                           cntb, tblv, zbi, zbf, obuf, bev, idxv,
                           b0, b1, b2, b3,
                           tok_sp, w_sp, tbl_sp,
                           g0, g1, g2, g3, s0, s1, s2, s3):
    cid = lax.axis_index("c")
    tid = lax.axis_index("s")
    wid = tid * 2 + cid
    kk0 = tid < NT // 2
    seg = lax.rem(tid, NT // 2)
    toff = seg * APT
    lane = lax.iota(jnp.int32, 16)
    stripe = pl.ds(tid * (NSLOTS // NT), NSLOTS // NT)
    bufs, gsem, ssem = (b0, b1, b2, b3), (g0, g1, g2, g3), (s0, s1, s2, s3)

    for i in range(NSLOTS // NT // 16):
        zbi[pl.ds(i * 16, 16)] = jnp.zeros((16,), jnp.int32)
        zbf[pl.ds(i * 16, 16)] = jnp.zeros((16,), jnp.float32)
    z0 = pltpu.async_copy(zbi, tok_sp.at[stripe], s0)
    z1 = pltpu.async_copy(zbf, w_sp.at[stripe], s1)

    @pl.when(kk0)
    def _():
        cp0 = pltpu.async_copy(e0_hbm.at[pl.ds(toff, APT)], ev0, s2)
        cp1 = pltpu.async_copy(w0_hbm.at[pl.ds(toff, APT)], wv0, s3)
        cp0.wait()
        cp1.wait()

    @pl.when(jnp.logical_not(kk0))
    def _():
        cp0 = pltpu.async_copy(e1_hbm.at[pl.ds(toff, APT)], ev0, s2)
        cp1 = pltpu.async_copy(w1_hbm.at[pl.ds(toff, APT)], wv0, s3)
        cp0.wait()
        cp1.wait()

    counts = jnp.zeros((16,), jnp.int32)
    cntb[...] = counts
    pltpu.sync_copy(cntb, tbl_sp.at[tid])
    z0.wait()
    z1.wait()
    plsc.subcore_barrier()

    pltpu.sync_copy(tbl_sp, tblv)
    counts_all = jnp.zeros((16,), jnp.int32)
    my_prefix = jnp.zeros((16,), jnp.int32)
    for i in range(NT):
        row = tblv[i]
        counts_all = counts_all + row
        my_prefix = my_prefix + _vsel(tid > i, row, jnp.zeros((16,), jnp.int32))
    padded = ((counts_all + (SBLK - 1)) >> 8) << 8
    start_excl = plsc.cumsum(padded) - padded
    base_cnt = start_excl + my_prefix

    cnt = [jnp.full((16,), jnp.sum(jnp.where(lane == ex, base_cnt, 0)),
                    jnp.int32) for ex in range(NE)]
    for i in range(APT // 16):
        sl = pl.ds(i * 16, 16)
        posb1d[sl] = jnp.zeros((16,), jnp.int32)
    aoff = jnp.where(kk0, 0, 2 * S // 2) + toff
    pp = pltpu.async_copy(posb1d, pos_hbm.at[pl.ds(aoff, APT)], g0)
    pp.wait()
    plsc.subcore_barrier()

    pltpu.sync_copy(tok_sp.at[pl.ds(wid * GPT, GPT)], idxv)
    _pipelined_gather(x_hbm, xs_hbm, idxv, bufs, gsem, ssem,
                      wid * GPT, GPT // GCH)

    @pl.when(cid == 0)
    def _():
        pltpu.sync_copy(w_sp.at[stripe], obuf)
        pltpu.sync_copy(obuf, sw_hbm.at[stripe])

    @pl.when((cid == 0) & (tid == 0))
    def _():
        for bi in range(NBPAD // 16):
            bvec = (bi * 16 + lane) * SBLK
            be = jnp.full((16,), -1, jnp.int32)
            for ex in range(NE):
                st = jnp.sum(jnp.where(lane == ex, start_excl, 0))
                be = be + jnp.where(bvec >= st, 1, 0)
            bev[pl.ds(bi * 16, 16)] = jnp.minimum(be, NE - 1)
        pltpu.sync_copy(bev, be_hbm)


def _combine_gather_body(y_hbm, pos_hbm, out_hbm, idxv,
                         b0, b1, b2, b3, g0, g1, g2, g3, s0, s1, s2, s3):
    wid = lax.axis_index("s") * 2 + lax.axis_index("c")
    pltpu.sync_copy(pos_hbm.at[pl.ds(wid * CPT, CPT)], idxv)
    _pipelined_gather(y_hbm, out_hbm, idxv,
                      (b0, b1, b2, b3), (g0, g1, g2, g3), (s0, s1, s2, s3),
                      wid * CPT, CPT // GCH)


def kernel(x, W_router, Wg, Wu, Wd, Wg_s, Wu_s, Wd_s):
    x_flat = x.reshape(S, DH)
    mesh = plsc.VectorSubcoreMesh(core_axis_name="c", subcore_axis_name="s")

    w0, w1, e0, e1 = pl.pallas_call(
        _router_kernel,
        grid=(S // RBLK,),
        in_specs=[
            pl.BlockSpec((RBLK, DH), lambda b: (b, 0)),
            pl.BlockSpec((NE, DH), lambda b: (0, 0)),
        ],
        out_specs=[pl.BlockSpec((RBLK,), lambda b: (b,))] * 4,
        out_shape=[
            jax.ShapeDtypeStruct((S,), jnp.float32),
            jax.ShapeDtypeStruct((S,), jnp.float32),
            jax.ShapeDtypeStruct((S,), jnp.int32),
            jax.ShapeDtypeStruct((S,), jnp.int32),
        ],
        interpret=_INTERPRET,
    )(x_flat, W_router)

    x_sorted, slot_w, pos01, block_expert = pl.kernel(
        _routing_dispatch_body,
        mesh=mesh,
        out_type=[
            jax.ShapeDtypeStruct((NSLOTS, DH), jnp.float32),
            jax.ShapeDtypeStruct((NSLOTS,), jnp.float32),
            jax.ShapeDtypeStruct((2 * S,), jnp.int32),
            jax.ShapeDtypeStruct((NBPAD,), jnp.int32),
        ],
        scratch_types=(
            [pltpu.VMEM((APT,), jnp.int32)]
            + [pltpu.VMEM((APT,), jnp.float32)]
            + [pltpu.VMEM((2, 128), jnp.int32)]
            + [pltpu.VMEM((APT,), jnp.int32)]
            + [pltpu.VMEM((2, 128), jnp.int32)]
            + [pltpu.VMEM((2, 128), jnp.float32)]
            + [pltpu.VMEM((16,), jnp.int32)]
            + [pltpu.VMEM((NT, 16), jnp.int32)]
            + [pltpu.VMEM((NSLOTS // NT,), jnp.int32)]
            + [pltpu.VMEM((NSLOTS // NT,), jnp.float32)]
            + [pltpu.VMEM((NSLOTS // NT,), jnp.float32)]
            + [pltpu.VMEM((NBPAD,), jnp.int32)]
            + [pltpu.VMEM((GPT,), jnp.int32)]
            + [pltpu.VMEM((GCH, DH), jnp.float32)] * 4
            + [pltpu.VMEM_SHARED((NSLOTS,), jnp.int32)]
            + [pltpu.VMEM_SHARED((NSLOTS,), jnp.float32)]
            + [pltpu.VMEM_SHARED((NT, 16), jnp.int32)]
            + [pltpu.SemaphoreType.DMA] * 8
        ),
        compiler_params=pltpu.CompilerParams(needs_layout_passes=False),
    )(x_flat, e0, e1, w0, w1)

    shared_out = pl.pallas_call(
        _shared_kernel,
        grid=(S // TBLK,),
        in_specs=[
            pl.BlockSpec((TBLK, DH), lambda b: (b, 0)),
            pl.BlockSpec((2 * DE, DH), lambda b: (0, 0)),
            pl.BlockSpec((2 * DE, DH), lambda b: (0, 0)),
            pl.BlockSpec((DH, 2 * DE), lambda b: (0, 0)),
        ],
        out_specs=pl.BlockSpec((TBLK, DH), lambda b: (b, 0)),
        out_shape=jax.ShapeDtypeStruct((S, DH), jnp.float32),
        interpret=_INTERPRET,
    )(x_flat, Wg_s, Wu_s, Wd_s)

    y_slots = pl.pallas_call(
        _expert_kernel,
        grid_spec=pltpu.PrefetchScalarGridSpec(
            num_scalar_prefetch=1,
            grid=(NB,),
            in_specs=[
                pl.BlockSpec((SBLK, DH), lambda b, be: (b, 0)),
                pl.BlockSpec((1, DE, DH), lambda b, be: (be[b], 0, 0)),
                pl.BlockSpec((1, DE, DH), lambda b, be: (be[b], 0, 0)),
                pl.BlockSpec((1, DH, DE), lambda b, be: (be[b], 0, 0)),
                pl.BlockSpec((SBLK, 1), lambda b, be: (b, 0)),
            ],
            out_specs=pl.BlockSpec((SBLK, DH), lambda b, be: (b, 0)),
        ),
        out_shape=jax.ShapeDtypeStruct((NSLOTS, DH), jnp.float32),
        interpret=_INTERPRET,
    )(block_expert, x_sorted, Wg, Wu, Wd, slot_w.reshape(NSLOTS, 1))

    yg = pl.kernel(
        _combine_gather_body,
        mesh=mesh,
        out_type=jax.ShapeDtypeStruct((2 * S, DH), jnp.float32),
        scratch_types=(
            [pltpu.VMEM((CPT,), jnp.int32)]
            + [pltpu.VMEM((GCH, DH), jnp.float32)] * 4
            + [pltpu.SemaphoreType.DMA] * 8
        ),
        compiler_params=pltpu.CompilerParams(needs_layout_passes=False),
    )(y_slots, pos01)

    out_flat = pl.pallas_call(
        _add3_kernel,
        grid=(S // 256,),
        in_specs=[
            pl.BlockSpec((256, DH), lambda b: (b, 0)),
            pl.BlockSpec((256, DH), lambda b: (b, 0)),
            pl.BlockSpec((256, DH), lambda b: (b + S // 256, 0)),
        ],
        out_specs=pl.BlockSpec((256, DH), lambda b: (b, 0)),
        out_shape=jax.ShapeDtypeStruct((S, DH), jnp.float32),
        interpret=_INTERPRET,
    )(shared_out, yg, yg)

    return out_flat.reshape(x.shape)


# speedup vs baseline: 1.9922x; 1.9922x over previous
import jax
import jax.numpy as jnp
from jax import lax
from jax.experimental import pallas as pl
from jax.experimental.pallas import tpu as pltpu
from jax.experimental.pallas import tpu_sc as plsc

S = 2048
DH = 2048
DE = 1024
NE = 8
RBLK = 512
TBLK = 128
SBLK = 256
NSLOTS = 2 * S + NE * SBLK
NB = NSLOTS // SBLK
NBPAD = 32
NW = 32
NT = 16
APT = 2 * S // NT
GPT = NSLOTS // NW
CPT = 2 * S // NW
GCH = 8

_INTERPRET = False


def _silu(v):
    return v * jax.nn.sigmoid(v)


def _vsel(pred_scalar, a, b):
    pi = jnp.full((16,), pred_scalar.astype(jnp.int32))
    return jnp.where(pi == 1, a, b)


def _router_kernel(x_ref, wr_ref, w0_ref, w1_ref, e0_ref, e1_ref):
    xb = x_ref[...]
    lg = lax.dot_general(xb, wr_ref[...], (((1,), (1,)), ((), ())),
                         preferred_element_type=jnp.float32)
    m = jnp.max(lg, axis=-1, keepdims=True)
    p = jnp.exp(lg - m)
    sc = p / jnp.sum(p, axis=-1, keepdims=True)
    iota = lax.broadcasted_iota(jnp.int32, (RBLK, NE), 1)
    s0 = jnp.max(sc, axis=-1)
    a0 = jnp.min(jnp.where(sc == s0[:, None], iota, NE), axis=-1)
    sc1 = jnp.where(iota == a0[:, None], -1.0, sc)
    s1 = jnp.max(sc1, axis=-1)
    a1 = jnp.min(jnp.where(sc1 == s1[:, None], iota, NE), axis=-1)
    w0_ref[...] = s0
    w1_ref[...] = s1
    e0_ref[...] = a0
    e1_ref[...] = a1


def _shared_kernel(x_ref, wg_ref, wu_ref, wd_ref, out_ref):
    xb = x_ref[...]
    g = lax.dot_general(xb, wg_ref[...], (((1,), (1,)), ((), ())),
                        preferred_element_type=jnp.float32)
    u = lax.dot_general(xb, wu_ref[...], (((1,), (1,)), ((), ())),
                        preferred_element_type=jnp.float32)
    gu = _silu(g) * u
    out_ref[...] = lax.dot_general(gu, wd_ref[...], (((1,), (1,)), ((), ())),
                                   preferred_element_type=jnp.float32)


def _expert_kernel(be_ref, x_ref, wg_ref, wu_ref, wd_ref, sw_ref, y_ref):
    del be_ref
    xb = x_ref[...]
    g = lax.dot_general(xb, wg_ref[0], (((1,), (1,)), ((), ())),
                        preferred_element_type=jnp.float32)
    u = lax.dot_general(xb, wu_ref[0], (((1,), (1,)), ((), ())),
                        preferred_element_type=jnp.float32)
    h = _silu(g) * u
    y = lax.dot_general(h, wd_ref[0], (((1,), (1,)), ((), ())),
                        preferred_element_type=jnp.float32)
    y_ref[...] = y * sw_ref[...]


def _add3_kernel(a_ref, b_ref, c_ref, o_ref):
    o_ref[...] = a_ref[...] + b_ref[...] + c_ref[...]


def _pipelined_gather(src_hbm, out_hbm, idx_v, bufs, gsem, ssem,
                      base, n_chunks):
    def st_wait(j, off):
        pltpu.make_async_copy(bufs[j], out_hbm.at[pl.ds(off, GCH)],
                              ssem[j]).wait()

    def quad(q, carry):
        offs = [base + (q * 4 + j) * GCH for j in range(4)]
        gathers = []
        for j in range(4):
            @pl.when(q > 0)
            def _(j=j, off=offs[j]):
                st_wait(j, off)
            gathers.append(pltpu.async_copy(
                src_hbm.at[idx_v.at[pl.ds((q * 4 + j) * GCH, GCH)]],
                bufs[j], gsem[j]))
        for j in range(4):
            gathers[j].wait()
            pltpu.async_copy(bufs[j], out_hbm.at[pl.ds(offs[j], GCH)],
                             ssem[j])
        return carry

    lax.fori_loop(0, n_chunks // 4, quad, 0)
    for j in range(4):
        st_wait(j, base + (n_chunks - 4 + j) * GCH)


def _routing_dispatch_body(x_hbm, e0_hbm, e1_hbm, w0_hbm, w1_hbm,
                           xs_hbm, sw_hbm, pos_hbm, be_hbm,
                           ev0, wv0, posb, posb1d, tokb, wb,
                           cntb, tblv, zbi, zbf, obuf, bev, idxv,
                           b0, b1, b2, b3,
                           tok_sp, w_sp, tbl_sp,
                           g0, g1, g2, g3, s0, s1, s2, s3):
    cid = lax.axis_index("c")
    tid = lax.axis_index("s")
    wid = tid * 2 + cid
    kk0 = tid < NT // 2
    seg = lax.rem(tid, NT // 2)
    toff = seg * APT
    lane = lax.iota(jnp.int32, 16)
    stripe = pl.ds(tid * (NSLOTS // NT), NSLOTS // NT)
    bufs, gsem, ssem = (b0, b1, b2, b3), (g0, g1, g2, g3), (s0, s1, s2, s3)

    for i in range(NSLOTS // NT // 16):
        zbi[pl.ds(i * 16, 16)] = jnp.zeros((16,), jnp.int32)
        zbf[pl.ds(i * 16, 16)] = jnp.zeros((16,), jnp.float32)
    z0 = pltpu.async_copy(zbi, tok_sp.at[stripe], s0)
    z1 = pltpu.async_copy(zbf, w_sp.at[stripe], s1)

    @pl.when(kk0)
    def _():
        cp0 = pltpu.async_copy(e0_hbm.at[pl.ds(toff, APT)], ev0, s2)
        cp1 = pltpu.async_copy(w0_hbm.at[pl.ds(toff, APT)], wv0, s3)
        cp0.wait()
        cp1.wait()

    @pl.when(jnp.logical_not(kk0))
    def _():
        cp0 = pltpu.async_copy(e1_hbm.at[pl.ds(toff, APT)], ev0, s2)
        cp1 = pltpu.async_copy(w1_hbm.at[pl.ds(toff, APT)], wv0, s3)
        cp0.wait()
        cp1.wait()

    counts = jnp.zeros((16,), jnp.int32)
    for i in range(APT // 16):
        sl = pl.ds(i * 16, 16)
        ve = ev0[sl]
        for ex in range(NE):
            pc = plsc.all_reduce_population_count(ve == ex)
            counts = counts + jnp.where(lane == ex, pc, 0)
    cntb[...] = counts
    pltpu.sync_copy(cntb, tbl_sp.at[tid])
    z0.wait()
    z1.wait()
    plsc.subcore_barrier()

    pltpu.sync_copy(tbl_sp, tblv)
    counts_all = jnp.zeros((16,), jnp.int32)
    my_prefix = jnp.zeros((16,), jnp.int32)
    for i in range(NT):
        row = tblv[i]
        counts_all = counts_all + row
        my_prefix = my_prefix + _vsel(tid > i, row, jnp.zeros((16,), jnp.int32))
    padded = ((counts_all + (SBLK - 1)) >> 8) << 8
    start_excl = plsc.cumsum(padded) - padded
    base_cnt = start_excl + my_prefix
    live_end = start_excl + counts_all
    pad_end = start_excl + padded

    cnt = [jnp.full((16,), jnp.sum(jnp.where(lane == ex, base_cnt, 0)),
                    jnp.int32) for ex in range(NE)]
    for i in range(APT // 16):
        sl = pl.ds(i * 16, 16)
        ve = ev0[sl]
        vw = wv0[sl]
        tokv = toff + i * 16 + lane
        pos_v = jnp.zeros((16,), jnp.int32)
        for ex in range(NE):
            m = ve == ex
            mi = jnp.where(m, 1, 0)
            cs = plsc.cumsum(mi)
            pos_v = pos_v + jnp.where(m, cnt[ex] + (cs - mi), 0)
            cnt[ex] = cnt[ex] + plsc.all_reduce_population_count(m)
        row, col = i // 8, (i % 8) * 16
        posb[row, pl.ds(col, 16)] = pos_v
        posb1d[sl] = pos_v
        tokb[row, pl.ds(col, 16)] = tokv
        wb[row, pl.ds(col, 16)] = vw
    sc0 = pltpu.async_copy(tokb.at[0], tok_sp.at[posb.at[0]], s0, add=True)
    sc1 = pltpu.async_copy(tokb.at[1], tok_sp.at[posb.at[1]], s1, add=True)
    sc2 = pltpu.async_copy(wb.at[0], w_sp.at[posb.at[0]], s2, add=True)
    sc3 = pltpu.async_copy(wb.at[1], w_sp.at[posb.at[1]], s3, add=True)
    aoff = jnp.where(kk0, 0, 2 * S // 2) + toff
    pp = pltpu.async_copy(posb1d, pos_hbm.at[pl.ds(aoff, APT)], g0)
    sc0.wait()
    sc1.wait()
    sc2.wait()
    sc3.wait()
    pp.wait()
    plsc.subcore_barrier()

    pltpu.sync_copy(tok_sp.at[pl.ds(wid * GPT, GPT)], idxv)
    les = [jnp.sum(jnp.where(lane == ex, live_end, 0)) for ex in range(NE)]
    pes = [jnp.sum(jnp.where(lane == ex, pad_end, 0)) for ex in range(NE)]
    for i in range(GPT // 16):
        isl = pl.ds(i * 16, 16)
        pvec = wid * GPT + i * 16 + lane
        dead = pvec >= pes[NE - 1]
        for ex in range(NE):
            dead = jnp.logical_or(
                dead, jnp.logical_and(pvec >= les[ex], pvec < pes[ex]))
        idxv[isl] = jnp.where(dead, pvec & (S - 1), idxv[isl])
    _pipelined_gather(x_hbm, xs_hbm, idxv, bufs, gsem, ssem,
                      wid * GPT, GPT // GCH)

    @pl.when(cid == 0)
    def _():
        pltpu.sync_copy(w_sp.at[stripe], obuf)
        pltpu.sync_copy(obuf, sw_hbm.at[stripe])

    @pl.when((cid == 0) & (tid == 0))
    def _():
        for bi in range(NBPAD // 16):
            bvec = (bi * 16 + lane) * SBLK
            be = jnp.full((16,), -1, jnp.int32)
            for ex in range(NE):
                st = jnp.sum(jnp.where(lane == ex, start_excl, 0))
                be = be + jnp.where(bvec >= st, 1, 0)
            bev[pl.ds(bi * 16, 16)] = jnp.minimum(be, NE - 1)
        pltpu.sync_copy(bev, be_hbm)


def _combine_gather_body(y_hbm, pos_hbm, out_hbm, idxv,
                         b0, b1, b2, b3, g0, g1, g2, g3, s0, s1, s2, s3):
    wid = lax.axis_index("s") * 2 + lax.axis_index("c")
    pltpu.sync_copy(pos_hbm.at[pl.ds(wid * CPT, CPT)], idxv)
    _pipelined_gather(y_hbm, out_hbm, idxv,
                      (b0, b1, b2, b3), (g0, g1, g2, g3), (s0, s1, s2, s3),
                      wid * CPT, CPT // GCH)


def kernel(x, W_router, Wg, Wu, Wd, Wg_s, Wu_s, Wd_s):
    x_flat = x.reshape(S, DH)
    mesh = plsc.VectorSubcoreMesh(core_axis_name="c", subcore_axis_name="s")

    w0, w1, e0, e1 = pl.pallas_call(
        _router_kernel,
        grid=(S // RBLK,),
        in_specs=[
            pl.BlockSpec((RBLK, DH), lambda b: (b, 0)),
            pl.BlockSpec((NE, DH), lambda b: (0, 0)),
        ],
        out_specs=[pl.BlockSpec((RBLK,), lambda b: (b,))] * 4,
        out_shape=[
            jax.ShapeDtypeStruct((S,), jnp.float32),
            jax.ShapeDtypeStruct((S,), jnp.float32),
            jax.ShapeDtypeStruct((S,), jnp.int32),
            jax.ShapeDtypeStruct((S,), jnp.int32),
        ],
        interpret=_INTERPRET,
    )(x_flat, W_router)

    x_sorted, slot_w, pos01, block_expert = pl.kernel(
        _routing_dispatch_body,
        mesh=mesh,
        out_type=[
            jax.ShapeDtypeStruct((NSLOTS, DH), jnp.float32),
            jax.ShapeDtypeStruct((NSLOTS,), jnp.float32),
            jax.ShapeDtypeStruct((2 * S,), jnp.int32),
            jax.ShapeDtypeStruct((NBPAD,), jnp.int32),
        ],
        scratch_types=(
            [pltpu.VMEM((APT,), jnp.int32)]
            + [pltpu.VMEM((APT,), jnp.float32)]
            + [pltpu.VMEM((2, 128), jnp.int32)]
            + [pltpu.VMEM((APT,), jnp.int32)]
            + [pltpu.VMEM((2, 128), jnp.int32)]
            + [pltpu.VMEM((2, 128), jnp.float32)]
            + [pltpu.VMEM((16,), jnp.int32)]
            + [pltpu.VMEM((NT, 16), jnp.int32)]
            + [pltpu.VMEM((NSLOTS // NT,), jnp.int32)]
            + [pltpu.VMEM((NSLOTS // NT,), jnp.float32)]
            + [pltpu.VMEM((NSLOTS // NT,), jnp.float32)]
            + [pltpu.VMEM((NBPAD,), jnp.int32)]
            + [pltpu.VMEM((GPT,), jnp.int32)]
            + [pltpu.VMEM((GCH, DH), jnp.float32)] * 4
            + [pltpu.VMEM_SHARED((NSLOTS,), jnp.int32)]
            + [pltpu.VMEM_SHARED((NSLOTS,), jnp.float32)]
            + [pltpu.VMEM_SHARED((NT, 16), jnp.int32)]
            + [pltpu.SemaphoreType.DMA] * 8
        ),
        compiler_params=pltpu.CompilerParams(needs_layout_passes=False),
    )(x_flat, e0, e1, w0, w1)

    shared_out = pl.pallas_call(
        _shared_kernel,
        grid=(S // TBLK,),
        in_specs=[
            pl.BlockSpec((TBLK, DH), lambda b: (b, 0)),
            pl.BlockSpec((2 * DE, DH), lambda b: (0, 0)),
            pl.BlockSpec((2 * DE, DH), lambda b: (0, 0)),
            pl.BlockSpec((DH, 2 * DE), lambda b: (0, 0)),
        ],
        out_specs=pl.BlockSpec((TBLK, DH), lambda b: (b, 0)),
        out_shape=jax.ShapeDtypeStruct((S, DH), jnp.float32),
        interpret=_INTERPRET,
    )(x_flat, Wg_s, Wu_s, Wd_s)

    y_slots = pl.pallas_call(
        _expert_kernel,
        grid_spec=pltpu.PrefetchScalarGridSpec(
            num_scalar_prefetch=1,
            grid=(NB,),
            in_specs=[
                pl.BlockSpec((SBLK, DH), lambda b, be: (b, 0)),
                pl.BlockSpec((1, DE, DH), lambda b, be: (be[b], 0, 0)),
                pl.BlockSpec((1, DE, DH), lambda b, be: (be[b], 0, 0)),
                pl.BlockSpec((1, DH, DE), lambda b, be: (be[b], 0, 0)),
                pl.BlockSpec((SBLK, 1), lambda b, be: (b, 0)),
            ],
            out_specs=pl.BlockSpec((SBLK, DH), lambda b, be: (b, 0)),
        ),
        out_shape=jax.ShapeDtypeStruct((NSLOTS, DH), jnp.float32),
        interpret=_INTERPRET,
    )(block_expert, x_sorted, Wg, Wu, Wd, slot_w.reshape(NSLOTS, 1))

    yg = pl.kernel(
        _combine_gather_body,
        mesh=mesh,
        out_type=jax.ShapeDtypeStruct((2 * S, DH), jnp.float32),
        scratch_types=(
            [pltpu.VMEM((CPT,), jnp.int32)]
            + [pltpu.VMEM((GCH, DH), jnp.float32)] * 4
            + [pltpu.SemaphoreType.DMA] * 8
        ),
        compiler_params=pltpu.CompilerParams(needs_layout_passes=False),
    )(y_slots, pos01)

    out_flat = pl.pallas_call(
        _add3_kernel,
        grid=(S // 256,),
        in_specs=[
            pl.BlockSpec((256, DH), lambda b: (b, 0)),
            pl.BlockSpec((256, DH), lambda b: (b, 0)),
            pl.BlockSpec((256, DH), lambda b: (b + S // 256, 0)),
        ],
        out_specs=pl.BlockSpec((256, DH), lambda b: (b, 0)),
        out_shape=jax.ShapeDtypeStruct((S, DH), jnp.float32),
        interpret=_INTERPRET,
    )(shared_out, yg, yg)

    return out_flat.reshape(x.shape)
